# Initial kernel scaffold; baseline (speedup 1.0000x reference)
import jax
import jax.numpy as jnp
from jax.experimental import pallas as pl


def kernel(x, batch, W1, b1, W2, b2):
    raise NotImplementedError("write your pallas kernel here")


# speedup vs baseline: 4.5689x; 4.5689x over previous
import functools

import jax
import jax.numpy as jnp
from jax import lax
from jax.experimental import pallas as pl
from jax.experimental.pallas import tpu as pltpu, tpu_sc as plsc

_N = 100000
_D = 128
_S = 512

_ROWS_TC = 1024
_GRID_TC = (_N + _ROWS_TC - 1) // _ROWS_TC
_NPAD = _GRID_TC * _ROWS_TC

_NTILES = 32
_SEG_PER_TILE = _S // _NTILES
_CHUNK = 256
_OFF_PAD = 528


def _tc_body(x_ref, w1_ref, b1_ref, w2t_ref, b2_ref, e_ref):
    h = jnp.tanh(
        jnp.dot(x_ref[...], w1_ref[...], preferred_element_type=jnp.float32)
        + b1_ref[...]
    )
    logit = jnp.sum(h * w2t_ref[...], axis=1) + b2_ref[0, 0]
    e_ref[...] = jnp.exp(logit).reshape(_ROWS_TC // _D, _D)


def _tc_weights(x, W1, b1, W2, b2):
    return pl.pallas_call(
        _tc_body,
        grid=(_GRID_TC,),
        in_specs=[
            pl.BlockSpec((_ROWS_TC, _D), lambda i: (i, 0)),
            pl.BlockSpec((_D, _D), lambda i: (0, 0)),
            pl.BlockSpec((1, _D), lambda i: (0, 0)),
            pl.BlockSpec((1, _D), lambda i: (0, 0)),
            pl.BlockSpec((1, 1), lambda i: (0, 0)),
        ],
        out_specs=pl.BlockSpec((_ROWS_TC // _D, _D), lambda i: (i, 0)),
        out_shape=jax.ShapeDtypeStruct((_NPAD // _D, _D), jnp.float32),
    )(x, W1, b1.reshape(1, _D), W2.reshape(1, _D), b2.reshape(1, 1))


def _sc_pool(x, e_flat, ids_pad, offsets):
    mesh = plsc.VectorSubcoreMesh(core_axis_name="c", subcore_axis_name="s")

    @functools.partial(
        pl.kernel,
        out_type=jax.ShapeDtypeStruct((_S, _D), jnp.float32),
        mesh=mesh,
        scratch_types=[
            pltpu.VMEM((_CHUNK + 8, _D), jnp.float32),
            pltpu.VMEM((_CHUNK + 24,), jnp.float32),
            pltpu.VMEM((_CHUNK + 24,), jnp.int32),
            pltpu.VMEM((_SEG_PER_TILE, _D), jnp.float32),
            pltpu.VMEM((16,), jnp.float32),
            pltpu.VMEM((16,), jnp.float32),
            pltpu.VMEM((_OFF_PAD,), jnp.int32),
        ],
    )
    def k(x_hbm, e_hbm, ids_hbm, off_hbm, out_hbm,
          xbuf, ebuf, idbuf, acc, den, scl, offv):
        wid = lax.axis_index("s") * 2 + lax.axis_index("c")
        seg0 = wid * _SEG_PER_TILE

        pltpu.sync_copy(off_hbm, offv)
        r0 = offv[pl.ds(seg0, 16)][0]
        r1 = offv[pl.ds(seg0 + _SEG_PER_TILE, 16)][0]

        zero = jnp.zeros((16,), jnp.float32)
        for j in range(_SEG_PER_TILE):
            for q in range(_D // 16):
                acc[j, pl.ds(16 * q, 16)] = zero
        den[...] = zero

        lane = lax.broadcasted_iota(jnp.int32, (16,), 0)
        nchunks = (r1 - r0 + _CHUNK - 1) // _CHUNK

        def chunk_body(kk, _):
            c0 = r0 + kk * _CHUNK
            c1 = jnp.minimum(c0 + _CHUNK, r1)
            sa = pl.multiple_of(
                jnp.minimum(
                    lax.bitwise_and(c0, jnp.int32(~7)), _N - (_CHUNK + 8)
                ),
                8,
            )
            pltpu.sync_copy(x_hbm.at[pl.ds(sa, _CHUNK + 8)], xbuf)
            pltpu.sync_copy(e_hbm.at[pl.ds(sa, _CHUNK + 24)], ebuf)
            pltpu.sync_copy(ids_hbm.at[pl.ds(sa, _CHUNK + 24)], idbuf)

            def row_body(r, _c):
                eb = ebuf[pl.ds(r - sa, 16)][0]
                sl = idbuf[pl.ds(r - sa, 16)][0] - seg0
                xo = r - sa
                for q in range(_D // 16):
                    plsc.addupdate(
                        acc.at[sl, pl.ds(16 * q, 16)],
                        xbuf[xo, pl.ds(16 * q, 16)] * eb,
                    )
                plsc.addupdate(den.at[pl.ds(0, 16)], jnp.where(lane == sl, eb, 0.0))
                return _c

            lax.fori_loop(c0, c1, row_body, 0, unroll=False)
            return _

        lax.fori_loop(0, nchunks, chunk_body, 0, unroll=False)

        scl[...] = 1.0 / (den[...] + 1e-16)
        sv = scl[...]
        for j in range(_SEG_PER_TILE):
            sj = sv[j]
            for q in range(_D // 16):
                acc[j, pl.ds(16 * q, 16)] = acc[j, pl.ds(16 * q, 16)] * sj
        pltpu.sync_copy(acc, out_hbm.at[pl.ds(seg0, _SEG_PER_TILE)])

    return k(x, e_flat, ids_pad, offsets)


def kernel(x, batch, W1, b1, W2, b2):
    e = _tc_weights(x, W1, b1, W2, b2).reshape(_NPAD)
    ids_pad = jnp.pad(batch, (0, _NPAD - _N))
    offsets = jnp.searchsorted(
        batch, jnp.arange(_OFF_PAD, dtype=jnp.int32), side="left"
    ).astype(jnp.int32)
    return _sc_pool(x, e, ids_pad, offsets)


# speedup vs baseline: 8.0485x; 1.7616x over previous
import functools

import jax
import jax.numpy as jnp
from jax import lax
from jax.experimental import pallas as pl
from jax.experimental.pallas import tpu as pltpu, tpu_sc as plsc

_N = 100000
_D = 128
_S = 512

_ROWS_TC = 1024
_GRID_TC = (_N + _ROWS_TC - 1) // _ROWS_TC
_NPAD = _GRID_TC * _ROWS_TC

_NTILES = 32
_SEG_PER_TILE = _S // _NTILES
_CHUNK = 384
_XB = _CHUNK + 8
_EB = _CHUNK + 24
_OFF_PAD = 528


def _tc_body(x_ref, w1_ref, b1_ref, w2t_ref, b2_ref, e_ref):
    h = jnp.tanh(
        jnp.dot(x_ref[...], w1_ref[...], preferred_element_type=jnp.float32)
        + b1_ref[...]
    )
    logit = jnp.sum(h * w2t_ref[...], axis=1) + b2_ref[0, 0]
    e_ref[...] = jnp.exp(logit).reshape(_ROWS_TC // _D, _D)


def _tc_weights(x, W1, b1, W2, b2):
    return pl.pallas_call(
        _tc_body,
        grid=(_GRID_TC,),
        in_specs=[
            pl.BlockSpec((_ROWS_TC, _D), lambda i: (i, 0)),
            pl.BlockSpec((_D, _D), lambda i: (0, 0)),
            pl.BlockSpec((1, _D), lambda i: (0, 0)),
            pl.BlockSpec((1, _D), lambda i: (0, 0)),
            pl.BlockSpec((1, 1), lambda i: (0, 0)),
        ],
        out_specs=pl.BlockSpec((_ROWS_TC // _D, _D), lambda i: (i, 0)),
        out_shape=jax.ShapeDtypeStruct((_NPAD // _D, _D), jnp.float32),
    )(x, W1, b1.reshape(1, _D), W2.reshape(1, _D), b2.reshape(1, 1))


def _sc_pool(x, e_flat, offsets):
    mesh = plsc.VectorSubcoreMesh(core_axis_name="c", subcore_axis_name="s")

    @functools.partial(
        pl.kernel,
        out_type=jax.ShapeDtypeStruct((_S, _D), jnp.float32),
        mesh=mesh,
        scratch_types=[
            pltpu.VMEM((_XB, _D), jnp.float32),
            pltpu.VMEM((_XB, _D), jnp.float32),
            pltpu.VMEM((_EB,), jnp.float32),
            pltpu.VMEM((_EB,), jnp.float32),
            pltpu.VMEM((_SEG_PER_TILE, _D), jnp.float32),
            pltpu.VMEM((16,), jnp.float32),
            pltpu.VMEM((16,), jnp.float32),
            pltpu.VMEM((_OFF_PAD,), jnp.int32),
            pltpu.SemaphoreType.DMA,
            pltpu.SemaphoreType.DMA,
            pltpu.SemaphoreType.DMA,
            pltpu.SemaphoreType.DMA,
        ],
    )
    def k(x_hbm, e_hbm, off_hbm, out_hbm,
          xbufA, xbufB, ebufA, ebufB, acc, den, scl, offv,
          sxA, seA, sxB, seB):
        wid = lax.axis_index("s") * 2 + lax.axis_index("c")
        seg0 = wid * _SEG_PER_TILE

        pltpu.sync_copy(off_hbm, offv)
        r0 = offv[pl.ds(seg0, 16)][0]
        r1 = offv[pl.ds(seg0 + _SEG_PER_TILE, 16)][0]

        zero = jnp.zeros((16,), jnp.float32)
        for j in range(_SEG_PER_TILE):
            for q in range(_D // 16):
                acc[j, pl.ds(16 * q, 16)] = zero
        den[...] = zero

        lane = lax.broadcasted_iota(jnp.int32, (16,), 0)

        def chunk_bounds(kk):
            c0 = jnp.minimum(r0 + kk * _CHUNK, r1)
            c1 = jnp.minimum(c0 + _CHUNK, r1)
            sa = pl.multiple_of(
                jnp.minimum(lax.bitwise_and(c0, jnp.int32(~7)), _N - _XB), 8
            )
            return c0, c1, sa

        def start(kk, xbuf, ebuf, sx, se):
            _, _, sa = chunk_bounds(kk)
            pltpu.async_copy(x_hbm.at[pl.ds(sa, _XB)], xbuf, sx)
            pltpu.async_copy(e_hbm.at[pl.ds(sa, _EB)], ebuf, se)

        def wait(kk, xbuf, ebuf, sx, se):
            _, _, sa = chunk_bounds(kk)
            pltpu.make_async_copy(x_hbm.at[pl.ds(sa, _XB)], xbuf, sx).wait()
            pltpu.make_async_copy(e_hbm.at[pl.ds(sa, _EB)], ebuf, se).wait()

        zeros9 = (zero,) * 9
        starts = offv[pl.ds(seg0, 16)]
        ends = offv[pl.ds(seg0 + 1, 16)]

        def process(kk, xbuf, ebuf):
            c0, c1, sa = chunk_bounds(kk)
            for j in range(_SEG_PER_TILE):
                lo = jnp.maximum(starts[j], c0)
                hi = jnp.minimum(ends[j], c1)

                def row_body(r, carr):
                    eb = ebuf[pl.ds(r - sa, 16)][0]
                    xo = r - sa
                    new = tuple(
                        carr[q] + xbuf[xo, pl.ds(16 * q, 16)] * eb
                        for q in range(8)
                    )
                    return new + (carr[8] + eb,)

                car = lax.fori_loop(lo, hi, row_body, zeros9)
                for q in range(8):
                    plsc.addupdate(acc.at[j, pl.ds(16 * q, 16)], car[q])
                plsc.addupdate(
                    den.at[pl.ds(0, 16)], jnp.where(lane == j, car[8], 0.0)
                )

        nch = (r1 - r0 + _CHUNK - 1) // _CHUNK
        npairs = jnp.maximum((nch + 1) // 2, 1)

        start(0, xbufA, ebufA, sxA, seA)

        def pair_body(i, _c):
            k0 = 2 * i
            wait(k0, xbufA, ebufA, sxA, seA)
            start(k0 + 1, xbufB, ebufB, sxB, seB)
            process(k0, xbufA, ebufA)
            wait(k0 + 1, xbufB, ebufB, sxB, seB)
            start(k0 + 2, xbufA, ebufA, sxA, seA)
            process(k0 + 1, xbufB, ebufB)
            return _c

        lax.fori_loop(0, npairs, pair_body, 0)
        wait(2 * npairs, xbufA, ebufA, sxA, seA)

        scl[...] = 1.0 / (den[...] + 1e-16)
        sv = scl[...]
        for j in range(_SEG_PER_TILE):
            sj = sv[j]
            for q in range(_D // 16):
                acc[j, pl.ds(16 * q, 16)] = acc[j, pl.ds(16 * q, 16)] * sj
        pltpu.sync_copy(acc, out_hbm.at[pl.ds(seg0, _SEG_PER_TILE)])

    return k(x, e_flat, offsets)


def kernel(x, batch, W1, b1, W2, b2):
    e = _tc_weights(x, W1, b1, W2, b2).reshape(_NPAD)
    offsets = jnp.searchsorted(
        batch, jnp.arange(_OFF_PAD, dtype=jnp.int32), side="left"
    ).astype(jnp.int32)
    return _sc_pool(x, e, offsets)
